# baseline (speedup 1.0000x reference)
import functools

import jax
import jax.numpy as jnp
from jax import lax
from jax.experimental import pallas as pl
from jax.experimental.pallas import tpu as pltpu
from jax.experimental.pallas import tpu_sc as plsc

CLASS_NUM = 100000
PART_NUM = 8
FEA_DIM = 32
BATCH = 16384
LAMBDA_C = 1.0
ROW = PART_NUM * FEA_DIM

NC = 2
NS = 16
NW = NC * NS
BPW = BATCH // NW
CH = 64
NCHUNK = BPW // CH
LANES = 16
VPR = ROW // LANES

TILE_C = 128
FT = FEA_DIM // 8
SC_TILE0 = 430
TPW = 11
W_C = TPW * TILE_C
NGRP = W_C // LANES
TAIL_VALID = CLASS_NUM - 781 * TILE_C
TAIL_OFF = (TPW - 1) * TILE_C + TAIL_VALID

_sc_mesh = plsc.VectorSubcoreMesh(core_axis_name="c", subcore_axis_name="s")


@functools.partial(
    pl.kernel,
    out_type=(
        jax.ShapeDtypeStruct((NW, LANES), jnp.float32),
        jax.ShapeDtypeStruct((NW, LANES), jnp.float32),
        jax.ShapeDtypeStruct((NW, LANES), jnp.float32),
    ),
    mesh=_sc_mesh,
    compiler_params=pltpu.CompilerParams(needs_layout_passes=False),
    scratch_types=[
        pltpu.VMEM((BPW,), jnp.int32),
        pltpu.VMEM((CH, ROW), jnp.float32),
        pltpu.VMEM((CH, ROW), jnp.float32),
        pltpu.VMEM((CH, ROW), jnp.float32),
        pltpu.VMEM((CH, ROW), jnp.float32),
        pltpu.VMEM((LANES,), jnp.float32),
        pltpu.VMEM((8, W_C), jnp.float32),
        pltpu.VMEM((8, W_C), jnp.float32),
        pltpu.VMEM((W_C,), jnp.float32),
        pltpu.SemaphoreType.DMA,
        pltpu.SemaphoreType.DMA,
        pltpu.SemaphoreType.DMA,
        pltpu.SemaphoreType.DMA,
        pltpu.SemaphoreType.DMA,
        pltpu.SemaphoreType.DMA,
    ],
)
def _sc_main(x_hbm, lbl_hbm, centers_hbm, ct_hbm, loss_out, s1_out, s2_out,
             idx_v, gb0, gb1, xb0, xb1, acc_v, sb0, sb1, gacc_v,
             sg0, sg1, sx0, sx1, sw0, sw1):
    wid = lax.axis_index("s") * NC + lax.axis_index("c")

    base = wid * BPW
    pltpu.sync_copy(lbl_hbm.at[pl.ds(base, BPW)], idx_v)

    gbufs = (gb0, gb1)
    xbufs = (xb0, xb1)
    gsems = (sg0, sg1)
    xsems = (sx0, sx1)

    def start(c):
        slot = c % 2
        cbase = c * CH
        g = pltpu.async_copy(
            centers_hbm.at[idx_v.at[pl.ds(cbase, CH)]], gbufs[slot], gsems[slot])
        x = pltpu.async_copy(
            x_hbm.at[pl.ds(base + cbase, CH)], xbufs[slot], xsems[slot])
        return g, x

    acc = jnp.zeros((LANES,), jnp.float32)
    pend = start(0)
    for c in range(NCHUNK):
        nxt = start(c + 1) if c + 1 < NCHUNK else None
        gcopy, xcopy = pend
        gcopy.wait()
        xcopy.wait()
        gb = gbufs[c % 2]
        xb = xbufs[c % 2]

        def row_body(r, a, gb=gb, xb=xb):
            for v in range(VPR):
                xv = xb[r, pl.ds(v * LANES, LANES)]
                gv = gb[r, pl.ds(v * LANES, LANES)]
                d = xv - gv
                a = a + d * d
            return a

        acc = lax.fori_loop(0, CH, row_body, acc)
        pend = nxt

    acc_v[...] = acc
    pltpu.sync_copy(acc_v, loss_out.at[wid])

    tbase = (SC_TILE0 + wid * TPW) * TILE_C
    sbufs = (sb0, sb1)
    ssems = (sw0, sw1)

    def plane_start(plane, slot):
        p = plane // FT
        ft = plane % FT
        pltpu.async_copy(
            ct_hbm.at[p, pl.ds(ft * 8, 8), pl.ds(tbase, W_C)],
            sbufs[slot], ssems[slot])

    def plane_wait(slot):
        pltpu.make_async_copy(
            ct_hbm.at[0, pl.ds(0, 8), pl.ds(tbase, W_C)],
            sbufs[slot], ssems[slot]).wait()

    def plane_accum(slot, first):
        buf = sbufs[slot]

        def grp(i, _, buf=buf):
            g = buf[0, pl.ds(i * LANES, LANES)]
            for f in range(1, 8):
                g = g + buf[f, pl.ds(i * LANES, LANES)]
            if first:
                gacc_v[pl.ds(i * LANES, LANES)] = g
            else:
                gacc_v[pl.ds(i * LANES, LANES)] = (
                    gacc_v[pl.ds(i * LANES, LANES)] + g)
            return 0

        lax.fori_loop(0, NGRP, grp, 0)

    zerov = jnp.zeros((LANES,), jnp.float32)
    s1a = jnp.zeros((LANES,), jnp.float32)
    s2a = jnp.zeros((LANES,), jnp.float32)
    plane_start(0, 0)
    for p in range(PART_NUM):
        for ft in range(FT):
            plane = p * FT + ft
            slot = plane % 2
            if plane + 1 < PART_NUM * FT:
                plane_start(plane + 1, (plane + 1) % 2)
            plane_wait(slot)
            plane_accum(slot, first=(ft == 0))

        @pl.when(wid == NW - 1)
        def _():
            for k in range((W_C - TAIL_OFF) // LANES):
                gacc_v[pl.ds(TAIL_OFF + k * LANES, LANES)] = zerov

        def fold(i, carry):
            a1, a2 = carry
            g = gacc_v[pl.ds(i * LANES, LANES)]
            return (a1 + g, a2 + g * g)

        s1a, s2a = lax.fori_loop(0, NGRP, fold, (s1a, s2a))

    acc_v[...] = s1a
    pltpu.sync_copy(acc_v, s1_out.at[wid])
    acc_v[...] = s2a
    pltpu.sync_copy(acc_v, s2_out.at[wid])


TC_C = SC_TILE0 * TILE_C
CB = 5504
NCB = TC_C // CB


def _stats_body(c_ref, s1_ref, s2_ref):
    blk = c_ref[...]
    g = jnp.sum(blk, axis=1)
    i = pl.program_id(0)
    j = pl.program_id(1)
    s1_ref[i, j] = jnp.sum(blk)
    s2_ref[i, j] = jnp.sum(g * g)


_stats_call = pl.pallas_call(
    _stats_body,
    grid=(PART_NUM, NCB),
    in_specs=[pl.BlockSpec((1, FEA_DIM, CB), lambda i, j: (i, 0, j))],
    out_specs=[
        pl.BlockSpec((PART_NUM, NCB), lambda i, j: (0, 0),
                     memory_space=pltpu.SMEM),
        pl.BlockSpec((PART_NUM, NCB), lambda i, j: (0, 0),
                     memory_space=pltpu.SMEM),
    ],
    out_shape=[
        jax.ShapeDtypeStruct((PART_NUM, NCB), jnp.float32),
        jax.ShapeDtypeStruct((PART_NUM, NCB), jnp.float32),
    ],
)


def kernel(x, label, centers):
    lbl = label.astype(jnp.int32)
    x2 = x.reshape(BATCH, ROW)
    c2 = centers.reshape(CLASS_NUM, ROW)
    ct = jnp.transpose(centers, (1, 2, 0))

    loss_p, s1sc, s2sc = _sc_main(x2, lbl, c2, ct)
    s1p, s2p = _stats_call(ct)

    n_all = CLASS_NUM * PART_NUM * FEA_DIM
    s1 = jnp.sum(s1p) + jnp.sum(s1sc)
    s2 = jnp.sum(s2p) + jnp.sum(s2sc)
    center_mean = s1 / n_all
    mean_m2 = s2 / (CLASS_NUM * PART_NUM * FEA_DIM * FEA_DIM)
    center_var = mean_m2 - center_mean * center_mean
    center_loss = LAMBDA_C * jnp.sum(loss_p) / (BATCH * PART_NUM * FEA_DIM)
    return (x, center_loss, center_mean, center_var)


# speedup vs baseline: 1.2352x; 1.2352x over previous
import functools

import jax
import jax.numpy as jnp
from jax import lax
from jax.experimental import pallas as pl
from jax.experimental.pallas import tpu as pltpu
from jax.experimental.pallas import tpu_sc as plsc

CLASS_NUM = 100000
PART_NUM = 8
FEA_DIM = 32
BATCH = 16384
LAMBDA_C = 1.0
ROW = PART_NUM * FEA_DIM

NC = 2
NS = 16
NW = NC * NS
BPW = BATCH // NW
CH = 64
NCHUNK = BPW // CH
LANES = 16
VPR = ROW // LANES

TILE_C = 128
FT = FEA_DIM // 8
SC_TILE0 = 46
TPW = 23
W_C = TPW * TILE_C
NGRP = W_C // LANES
TAIL_VALID = CLASS_NUM - 781 * TILE_C
TAIL_OFF = (TPW - 1) * TILE_C + TAIL_VALID

_sc_mesh = plsc.VectorSubcoreMesh(core_axis_name="c", subcore_axis_name="s")


@functools.partial(
    pl.kernel,
    out_type=(
        jax.ShapeDtypeStruct((NW, LANES), jnp.float32),
        jax.ShapeDtypeStruct((NW, LANES), jnp.float32),
    ),
    mesh=_sc_mesh,
    compiler_params=pltpu.CompilerParams(needs_layout_passes=False),
    scratch_types=[
        pltpu.VMEM((8, W_C), jnp.float32),
        pltpu.VMEM((8, W_C), jnp.float32),
        pltpu.VMEM((W_C,), jnp.float32),
        pltpu.VMEM((LANES,), jnp.float32),
        pltpu.SemaphoreType.DMA,
        pltpu.SemaphoreType.DMA,
    ],
)
def _sc_stats(ct_hbm, s1_out, s2_out,
              sb0, sb1, gacc_v, out_v, sw0, sw1):
    wid = lax.axis_index("s") * NC + lax.axis_index("c")
    tbase = (SC_TILE0 + wid * TPW) * TILE_C
    sbufs = (sb0, sb1)
    ssems = (sw0, sw1)

    def plane_start(plane, slot):
        p = plane // FT
        ft = plane % FT
        pltpu.async_copy(
            ct_hbm.at[p, pl.ds(ft * 8, 8), pl.ds(tbase, W_C)],
            sbufs[slot], ssems[slot])

    def plane_wait(slot):
        pltpu.make_async_copy(
            ct_hbm.at[0, pl.ds(0, 8), pl.ds(tbase, W_C)],
            sbufs[slot], ssems[slot]).wait()

    def plane_accum(slot, first):
        buf = sbufs[slot]

        def grp(i, _, buf=buf):
            g = buf[0, pl.ds(i * LANES, LANES)]
            for f in range(1, 8):
                g = g + buf[f, pl.ds(i * LANES, LANES)]
            if first:
                gacc_v[pl.ds(i * LANES, LANES)] = g
            else:
                gacc_v[pl.ds(i * LANES, LANES)] = (
                    gacc_v[pl.ds(i * LANES, LANES)] + g)
            return 0

        lax.fori_loop(0, NGRP, grp, 0)

    zerov = jnp.zeros((LANES,), jnp.float32)
    s1a = jnp.zeros((LANES,), jnp.float32)
    s2a = jnp.zeros((LANES,), jnp.float32)
    plane_start(0, 0)
    for p in range(PART_NUM):
        for ft in range(FT):
            plane = p * FT + ft
            slot = plane % 2
            if plane + 1 < PART_NUM * FT:
                plane_start(plane + 1, (plane + 1) % 2)
            plane_wait(slot)
            plane_accum(slot, first=(ft == 0))

        @pl.when(wid == NW - 1)
        def _():
            for k in range((W_C - TAIL_OFF) // LANES):
                gacc_v[pl.ds(TAIL_OFF + k * LANES, LANES)] = zerov

        def fold(i, carry):
            a1, a2 = carry
            g = gacc_v[pl.ds(i * LANES, LANES)]
            return (a1 + g, a2 + g * g)

        s1a, s2a = lax.fori_loop(0, NGRP, fold, (s1a, s2a))

    out_v[...] = s1a
    pltpu.sync_copy(out_v, s1_out.at[wid])
    out_v[...] = s2a
    pltpu.sync_copy(out_v, s2_out.at[wid])


@functools.partial(
    pl.kernel,
    out_type=jax.ShapeDtypeStruct((NW, LANES), jnp.float32),
    mesh=_sc_mesh,
    compiler_params=pltpu.CompilerParams(needs_layout_passes=False),
    scratch_types=[
        pltpu.VMEM((BPW,), jnp.int32),
        pltpu.VMEM((CH, ROW), jnp.float32),
        pltpu.VMEM((CH, ROW), jnp.float32),
        pltpu.VMEM((CH, ROW), jnp.float32),
        pltpu.VMEM((CH, ROW), jnp.float32),
        pltpu.VMEM((LANES,), jnp.float32),
        pltpu.SemaphoreType.DMA,
        pltpu.SemaphoreType.DMA,
        pltpu.SemaphoreType.DMA,
        pltpu.SemaphoreType.DMA,
    ],
)
def _sc_loss(x_hbm, lbl_hbm, centers_hbm, loss_out,
             idx_v, gb0, gb1, xb0, xb1, acc_v,
             sg0, sg1, sx0, sx1):
    wid = lax.axis_index("s") * NC + lax.axis_index("c")
    base = wid * BPW
    pltpu.sync_copy(lbl_hbm.at[pl.ds(base, BPW)], idx_v)

    gbufs = (gb0, gb1)
    xbufs = (xb0, xb1)
    gsems = (sg0, sg1)
    xsems = (sx0, sx1)

    def start(c):
        slot = c % 2
        cbase = c * CH
        g = pltpu.async_copy(
            centers_hbm.at[idx_v.at[pl.ds(cbase, CH)]], gbufs[slot], gsems[slot])
        x = pltpu.async_copy(
            x_hbm.at[pl.ds(base + cbase, CH)], xbufs[slot], xsems[slot])
        return g, x

    acc = jnp.zeros((LANES,), jnp.float32)
    pend = start(0)
    for c in range(NCHUNK):
        nxt = start(c + 1) if c + 1 < NCHUNK else None
        gcopy, xcopy = pend
        gcopy.wait()
        xcopy.wait()
        gb = gbufs[c % 2]
        xb = xbufs[c % 2]

        def row_body(r, a, gb=gb, xb=xb):
            for v in range(VPR):
                xv = xb[r, pl.ds(v * LANES, LANES)]
                gv = gb[r, pl.ds(v * LANES, LANES)]
                d = xv - gv
                a = a + d * d
            return a

        acc = lax.fori_loop(0, CH, row_body, acc)
        pend = nxt

    acc_v[...] = acc
    pltpu.sync_copy(acc_v, loss_out.at[wid])


TC_C = SC_TILE0 * TILE_C


def _stats_body(c_ref, s1_ref, s2_ref):
    blk = c_ref[...]
    g = jnp.sum(blk, axis=1)
    i = pl.program_id(0)
    s1_ref[i, 0] = jnp.sum(blk)
    s2_ref[i, 0] = jnp.sum(g * g)


_stats_call = pl.pallas_call(
    _stats_body,
    grid=(PART_NUM,),
    in_specs=[pl.BlockSpec((1, FEA_DIM, TC_C), lambda i: (i, 0, 0))],
    out_specs=[
        pl.BlockSpec((PART_NUM, 1), lambda i: (0, 0), memory_space=pltpu.SMEM),
        pl.BlockSpec((PART_NUM, 1), lambda i: (0, 0), memory_space=pltpu.SMEM),
    ],
    out_shape=[
        jax.ShapeDtypeStruct((PART_NUM, 1), jnp.float32),
        jax.ShapeDtypeStruct((PART_NUM, 1), jnp.float32),
    ],
)


def kernel(x, label, centers):
    lbl = label.astype(jnp.int32)
    x2 = x.reshape(BATCH, ROW)
    c2 = centers.reshape(CLASS_NUM, ROW)
    ct = jnp.transpose(centers, (1, 2, 0))

    s1sc, s2sc = _sc_stats(ct)
    s1p, s2p = _stats_call(ct)
    loss_p = _sc_loss(x2, lbl, c2)

    n_all = CLASS_NUM * PART_NUM * FEA_DIM
    s1 = jnp.sum(s1p) + jnp.sum(s1sc)
    s2 = jnp.sum(s2p) + jnp.sum(s2sc)
    center_mean = s1 / n_all
    mean_m2 = s2 / (CLASS_NUM * PART_NUM * FEA_DIM * FEA_DIM)
    center_var = mean_m2 - center_mean * center_mean
    center_loss = LAMBDA_C * jnp.sum(loss_p) / (BATCH * PART_NUM * FEA_DIM)
    return (x, center_loss, center_mean, center_var)


# speedup vs baseline: 1.2421x; 1.0056x over previous
import functools

import jax
import jax.numpy as jnp
from jax import lax
from jax.experimental import pallas as pl
from jax.experimental.pallas import tpu as pltpu
from jax.experimental.pallas import tpu_sc as plsc

CLASS_NUM = 100000
PART_NUM = 8
FEA_DIM = 32
BATCH = 16384
LAMBDA_C = 1.0
ROW = PART_NUM * FEA_DIM

NC = 2
NS = 16
NW = NC * NS
BPW = BATCH // NW
CH = 64
NCHUNK = BPW // CH
LANES = 16
VPR = ROW // LANES

TILE_C = 128
FT = FEA_DIM // 8
SC_TILE0 = 46
TPW = 23
W_C = TPW * TILE_C
NGRP = W_C // LANES
TAIL_VALID = CLASS_NUM - 781 * TILE_C
TAIL_OFF = (TPW - 1) * TILE_C + TAIL_VALID

_sc_mesh = plsc.VectorSubcoreMesh(core_axis_name="c", subcore_axis_name="s")


@functools.partial(
    pl.kernel,
    out_type=(
        jax.ShapeDtypeStruct((NW, LANES), jnp.float32),
        jax.ShapeDtypeStruct((NW, LANES), jnp.float32),
    ),
    mesh=_sc_mesh,
    compiler_params=pltpu.CompilerParams(needs_layout_passes=False),
    scratch_types=[
        pltpu.VMEM((8, W_C), jnp.float32),
        pltpu.VMEM((8, W_C), jnp.float32),
        pltpu.VMEM((W_C,), jnp.float32),
        pltpu.VMEM((LANES,), jnp.float32),
        pltpu.SemaphoreType.DMA,
        pltpu.SemaphoreType.DMA,
    ],
)
def _sc_stats(ct_hbm, x2_hbm, s1_out, s2_out,
              sb0, sb1, gacc_v, out_v, sw0, sw1):
    wid = lax.axis_index("s") * NC + lax.axis_index("c")
    pltpu.sync_copy(x2_hbm.at[pl.ds(0, 8), pl.ds(0, LANES)],
                    sb0.at[pl.ds(0, 8), pl.ds(0, LANES)])
    tbase = (SC_TILE0 + wid * TPW) * TILE_C
    sbufs = (sb0, sb1)
    ssems = (sw0, sw1)

    def plane_start(plane, slot):
        p = plane // FT
        ft = plane % FT
        pltpu.async_copy(
            ct_hbm.at[p, pl.ds(ft * 8, 8), pl.ds(tbase, W_C)],
            sbufs[slot], ssems[slot])

    def plane_wait(slot):
        pltpu.make_async_copy(
            ct_hbm.at[0, pl.ds(0, 8), pl.ds(tbase, W_C)],
            sbufs[slot], ssems[slot]).wait()

    def plane_accum(slot, first):
        buf = sbufs[slot]

        def grp(i, _, buf=buf):
            g = buf[0, pl.ds(i * LANES, LANES)]
            for f in range(1, 8):
                g = g + buf[f, pl.ds(i * LANES, LANES)]
            if first:
                gacc_v[pl.ds(i * LANES, LANES)] = g
            else:
                gacc_v[pl.ds(i * LANES, LANES)] = (
                    gacc_v[pl.ds(i * LANES, LANES)] + g)
            return 0

        lax.fori_loop(0, NGRP, grp, 0)

    zerov = jnp.zeros((LANES,), jnp.float32)
    s1a = jnp.zeros((LANES,), jnp.float32)
    s2a = jnp.zeros((LANES,), jnp.float32)
    plane_start(0, 0)
    for p in range(PART_NUM):
        for ft in range(FT):
            plane = p * FT + ft
            slot = plane % 2
            if plane + 1 < PART_NUM * FT:
                plane_start(plane + 1, (plane + 1) % 2)
            plane_wait(slot)
            plane_accum(slot, first=(ft == 0))

        @pl.when(wid == NW - 1)
        def _():
            for k in range((W_C - TAIL_OFF) // LANES):
                gacc_v[pl.ds(TAIL_OFF + k * LANES, LANES)] = zerov

        def fold(i, carry):
            a1, a2 = carry
            g = gacc_v[pl.ds(i * LANES, LANES)]
            return (a1 + g, a2 + g * g)

        s1a, s2a = lax.fori_loop(0, NGRP, fold, (s1a, s2a))

    out_v[...] = s1a
    pltpu.sync_copy(out_v, s1_out.at[wid])
    out_v[...] = s2a
    pltpu.sync_copy(out_v, s2_out.at[wid])


@functools.partial(
    pl.kernel,
    out_type=jax.ShapeDtypeStruct((NW, LANES), jnp.float32),
    mesh=_sc_mesh,
    compiler_params=pltpu.CompilerParams(needs_layout_passes=False),
    scratch_types=[
        pltpu.VMEM((BPW,), jnp.int32),
        pltpu.VMEM((CH, ROW), jnp.float32),
        pltpu.VMEM((CH, ROW), jnp.float32),
        pltpu.VMEM((CH, ROW), jnp.float32),
        pltpu.VMEM((CH, ROW), jnp.float32),
        pltpu.VMEM((LANES,), jnp.float32),
        pltpu.SemaphoreType.DMA,
        pltpu.SemaphoreType.DMA,
        pltpu.SemaphoreType.DMA,
        pltpu.SemaphoreType.DMA,
    ],
)
def _sc_loss(x_hbm, lbl_hbm, centers_hbm, loss_out,
             idx_v, gb0, gb1, xb0, xb1, acc_v,
             sg0, sg1, sx0, sx1):
    wid = lax.axis_index("s") * NC + lax.axis_index("c")
    base = wid * BPW
    pltpu.sync_copy(lbl_hbm.at[pl.ds(base, BPW)], idx_v)

    gbufs = (gb0, gb1)
    xbufs = (xb0, xb1)
    gsems = (sg0, sg1)
    xsems = (sx0, sx1)

    def start(c):
        slot = c % 2
        cbase = c * CH
        g = pltpu.async_copy(
            centers_hbm.at[idx_v.at[pl.ds(cbase, CH)]], gbufs[slot], gsems[slot])
        x = pltpu.async_copy(
            x_hbm.at[pl.ds(base + cbase, CH)], xbufs[slot], xsems[slot])
        return g, x

    acc = jnp.zeros((LANES,), jnp.float32)
    pend = start(0)
    for c in range(NCHUNK):
        nxt = start(c + 1) if c + 1 < NCHUNK else None
        gcopy, xcopy = pend
        gcopy.wait()
        xcopy.wait()
        gb = gbufs[c % 2]
        xb = xbufs[c % 2]

        def row_body(r, a, gb=gb, xb=xb):
            for v in range(VPR):
                xv = xb[r, pl.ds(v * LANES, LANES)]
                gv = gb[r, pl.ds(v * LANES, LANES)]
                d = xv - gv
                a = a + d * d
            return a

        acc = lax.fori_loop(0, CH, row_body, acc)
        pend = nxt

    acc_v[...] = acc
    pltpu.sync_copy(acc_v, loss_out.at[wid])


TC_C = SC_TILE0 * TILE_C


def _stats_body(c_ref, s1_ref, s2_ref):
    blk = c_ref[...]
    g = jnp.sum(blk, axis=1)
    i = pl.program_id(0)
    s1_ref[i, 0] = jnp.sum(blk)
    s2_ref[i, 0] = jnp.sum(g * g)


_stats_call = pl.pallas_call(
    _stats_body,
    grid=(PART_NUM,),
    in_specs=[pl.BlockSpec((1, FEA_DIM, TC_C), lambda i: (i, 0, 0))],
    out_specs=[
        pl.BlockSpec((PART_NUM, 1), lambda i: (0, 0), memory_space=pltpu.SMEM),
        pl.BlockSpec((PART_NUM, 1), lambda i: (0, 0), memory_space=pltpu.SMEM),
    ],
    out_shape=[
        jax.ShapeDtypeStruct((PART_NUM, 1), jnp.float32),
        jax.ShapeDtypeStruct((PART_NUM, 1), jnp.float32),
    ],
)


def kernel(x, label, centers):
    lbl = label.astype(jnp.int32)
    x2 = x.reshape(BATCH, ROW)
    c2 = centers.reshape(CLASS_NUM, ROW)
    ct = jnp.transpose(centers, (1, 2, 0))

    s1sc, s2sc = _sc_stats(ct, x2)
    s1p, s2p = _stats_call(ct)
    loss_p = _sc_loss(x2, lbl, c2)

    n_all = CLASS_NUM * PART_NUM * FEA_DIM
    s1 = jnp.sum(s1p) + jnp.sum(s1sc)
    s2 = jnp.sum(s2p) + jnp.sum(s2sc)
    center_mean = s1 / n_all
    mean_m2 = s2 / (CLASS_NUM * PART_NUM * FEA_DIM * FEA_DIM)
    center_var = mean_m2 - center_mean * center_mean
    center_loss = LAMBDA_C * jnp.sum(loss_p) / (BATCH * PART_NUM * FEA_DIM)
    return (x, center_loss, center_mean, center_var)
